# baseline (speedup 1.0000x reference)
import numpy as np
import jax
import jax.numpy as jnp
from jax.experimental import pallas as pl
from jax.experimental.pallas import tpu as pltpu

_VMEM_LIMIT = 60 * 1024 * 1024
_BB = 64


def _sel(kw, n_in, n_out, step):
    s = np.zeros((n_in, n_out), np.float32)
    for w in range(n_out):
        p = w + kw - 1
        if 0 <= p < n_in // step:
            s[step * p, w] = 1.0
    return s


def _toeplitz(w_taps, n_in, n_out, step, cw_rows=False):
    cin, cout = w_taps.shape[1], w_taps.shape[2]
    wt = w_taps.reshape(3, 3, cin, cout)
    s = jnp.stack([jnp.asarray(_sel(kw, n_in, n_out, step))
                   for kw in range(3)])
    if cw_rows:
        t = jnp.einsum('kuw,hkcd->hcuwd', s, wt)
    else:
        t = jnp.einsum('kuw,hkcd->hucwd', s, wt)
    return t.reshape(3, n_in * cin, n_out * cout).astype(jnp.bfloat16)


def _net_kernel(x_ref, t1_ref, b1_ref, t2_ref, b2_ref, t3_ref, b3_ref,
                fw1_ref, fb1_ref, fw2_ref, fb2_ref, o_ref):
    Bb = x_ref.shape[1]

    def conv(xp, t_ref, H):
        L = xp.shape[2]
        acc = None
        for kh in range(3):
            lhs = xp[kh:kh + H].reshape(H * Bb, L)
            d = jnp.dot(lhs, t_ref[kh], preferred_element_type=jnp.float32)
            acc = d if acc is None else acc + d
        return acc.reshape(H, Bb, t_ref.shape[2])

    def pool_bias_relu(y, blk, bt):
        y4 = y.reshape(y.shape[0] // 2, 2, Bb, y.shape[2])
        yh = jnp.maximum(y4[:, 0], y4[:, 1])
        yw = jnp.maximum(yh, jnp.roll(yh, -blk, axis=-1))
        return jnp.maximum(yw + bt, 0.0).astype(jnp.bfloat16)

    def padh(y):
        return jnp.pad(y, ((1, 1), (0, 0), (0, 0)))

    b1 = b1_ref[...].reshape(1, 1, -1)
    b2 = b2_ref[...].reshape(1, 1, -1)
    b3 = b3_ref[...].reshape(1, 1, -1)

    h = conv(x_ref[...], t1_ref, 32)
    h = pool_bias_relu(h, 32, b1)
    h = conv(padh(h), t2_ref, 16)
    h = pool_bias_relu(h, 64, b2)
    h = conv(padh(h), t3_ref, 8)
    h = pool_bias_relu(h, 128, b3)

    flat = jnp.concatenate([h[0], h[1], h[2], h[3]], axis=-1)
    z = jnp.dot(flat, fw1_ref[...], preferred_element_type=jnp.float32)
    z = jnp.maximum(z + fb1_ref[...], 0.0).astype(jnp.bfloat16)
    o_ref[...] = jnp.dot(z, fw2_ref[...],
                         preferred_element_type=jnp.float32) + fb2_ref[...]


def kernel(x_nchw, conv1_w, conv1_b, conv2_w, conv2_b, conv3_w, conv3_b,
           fc1_w, fc1_b, fc2_w, fc2_b):
    B = x_nchw.shape[0]
    xm = jnp.transpose(x_nchw, (2, 0, 1, 3)).astype(jnp.bfloat16)
    xm = jnp.pad(xm, ((1, 1), (0, 0), (0, 8 - xm.shape[2]), (0, 0)))
    xm = xm.reshape(34, B, 256)

    Bb = _BB
    Bpad = -B % Bb
    if Bpad:
        xm = jnp.pad(xm, ((0, 0), (0, Bpad), (0, 0)))
    G = (B + Bpad) // Bb

    t1 = _toeplitz(conv1_w, 32, 32, 1, cw_rows=True)
    t2 = _toeplitz(conv2_w, 32, 16, 2)
    t3 = _toeplitz(conv3_w, 16, 8, 2)
    b1t = jnp.tile(conv1_b, (1, 32))
    b2t = jnp.tile(conv2_b, (1, 16))
    b3t = jnp.tile(conv3_b, (1, 8))

    fw1 = jnp.zeros((4, 8, 128, 256), jnp.float32)
    fw1 = fw1.at[:, ::2, :, :].set(fc1_w.reshape(4, 4, 128, 256))
    fw1 = fw1.reshape(4096, 256).astype(jnp.bfloat16)
    fw2 = fc2_w.astype(jnp.bfloat16)

    out = pl.pallas_call(
        _net_kernel,
        out_shape=jax.ShapeDtypeStruct((B + Bpad, fw2.shape[1]), jnp.float32),
        grid=(G,),
        in_specs=[
            pl.BlockSpec((34, Bb, 256), lambda i: (0, i, 0)),
            pl.BlockSpec(t1.shape, lambda i: (0, 0, 0)),
            pl.BlockSpec(b1t.shape, lambda i: (0, 0)),
            pl.BlockSpec(t2.shape, lambda i: (0, 0, 0)),
            pl.BlockSpec(b2t.shape, lambda i: (0, 0)),
            pl.BlockSpec(t3.shape, lambda i: (0, 0, 0)),
            pl.BlockSpec(b3t.shape, lambda i: (0, 0)),
            pl.BlockSpec(fw1.shape, lambda i: (0, 0)),
            pl.BlockSpec(fc1_b.shape, lambda i: (0, 0)),
            pl.BlockSpec(fw2.shape, lambda i: (0, 0)),
            pl.BlockSpec(fc2_b.shape, lambda i: (0, 0)),
        ],
        out_specs=pl.BlockSpec((Bb, fw2.shape[1]), lambda i: (i, 0)),
        compiler_params=pltpu.CompilerParams(
            dimension_semantics=("parallel",),
            vmem_limit_bytes=_VMEM_LIMIT),
    )(xm, t1, b1t, t2, b2t, t3, b3t, fw1, fc1_b, fw2, fc2_b)
    return out[:B, :10]


# speedup vs baseline: 1.4873x; 1.4873x over previous
import numpy as np
import jax
import jax.numpy as jnp
from jax.experimental import pallas as pl
from jax.experimental.pallas import tpu as pltpu

_VMEM_LIMIT = 60 * 1024 * 1024
_BB = 64


def _sel(kw, n_in, n_out):
    s = np.zeros((n_in, n_out), np.float32)
    for w in range(n_out):
        p = w + kw - 1
        if 0 <= p < n_in:
            s[p, w] = 1.0
    return s


def _toeplitz(w_taps, n_in, n_out, cw_rows=False):
    cin, cout = w_taps.shape[1], w_taps.shape[2]
    wt = w_taps.reshape(3, 3, cin, cout)
    s = jnp.stack([jnp.asarray(_sel(kw, n_in, n_out))
                   for kw in range(3)])
    if cw_rows:
        t = jnp.einsum('kuw,hkcd->hcuwd', s, wt)
    else:
        t = jnp.einsum('kuw,hkcd->hucwd', s, wt)
    t = t.reshape(3 * n_in * cin, n_out, cout)
    perm = np.concatenate([np.arange(0, n_out, 2), np.arange(1, n_out, 2)])
    t = t[:, perm, :]
    return t.reshape(3 * n_in * cin, n_out * cout).astype(jnp.bfloat16)


def _net_kernel(x_ref, t1_ref, b1_ref, t2_ref, b2_ref, t3_ref, b3_ref,
                fw1_ref, fb1_ref, fw2_ref, fb2_ref, o_ref):
    Bb = x_ref.shape[1]

    def conv(xp, t_ref, H):
        lhs = jnp.concatenate([xp[0:H], xp[1:H + 1], xp[2:H + 2]], axis=-1)
        acc = jnp.dot(lhs.reshape(H * Bb, lhs.shape[2]), t_ref[...],
                      preferred_element_type=jnp.float32)
        return acc.reshape(H, Bb, t_ref.shape[1])

    def pool_bias_relu_compact(y, bt):
        y4 = y.reshape(y.shape[0] // 2, 2, Bb, y.shape[2])
        yh = jnp.maximum(y4[:, 0], y4[:, 1])
        half = yh.shape[2] // 2
        yw = jnp.maximum(yh[..., :half], yh[..., half:])
        return jnp.maximum(yw + bt, 0.0).astype(jnp.bfloat16)

    b1 = b1_ref[...].reshape(1, 1, -1)
    b2 = b2_ref[...].reshape(1, 1, -1)
    b3 = b3_ref[...].reshape(1, 1, -1)

    h = conv(x_ref[...], t1_ref, 32)
    h = pool_bias_relu_compact(h, b1)
    h = conv(jnp.pad(h, ((1, 1), (0, 0), (0, 0))), t2_ref, 16)
    h = pool_bias_relu_compact(h, b2)
    h = conv(jnp.pad(h, ((1, 1), (0, 0), (0, 0))), t3_ref, 8)
    h = pool_bias_relu_compact(h, b3)

    flat = jnp.concatenate([h[0], h[1], h[2], h[3]], axis=-1)
    z = jnp.dot(flat, fw1_ref[...], preferred_element_type=jnp.float32)
    z = jnp.maximum(z + fb1_ref[...], 0.0).astype(jnp.bfloat16)
    o_ref[...] = jnp.dot(z, fw2_ref[...],
                         preferred_element_type=jnp.float32) + fb2_ref[...]


def kernel(x_nchw, conv1_w, conv1_b, conv2_w, conv2_b, conv3_w, conv3_b,
           fc1_w, fc1_b, fc2_w, fc2_b):
    B = x_nchw.shape[0]
    xm = jnp.transpose(x_nchw, (2, 0, 1, 3)).astype(jnp.bfloat16)
    xm = jnp.pad(xm, ((1, 1), (0, 0), (0, 8 - xm.shape[2]), (0, 0)))
    xm = xm.reshape(34, B, 256)

    Bb = _BB
    Bpad = -B % Bb
    if Bpad:
        xm = jnp.pad(xm, ((0, 0), (0, Bpad), (0, 0)))
    G = (B + Bpad) // Bb

    t1 = _toeplitz(conv1_w, 32, 32, cw_rows=True)
    t2 = _toeplitz(conv2_w, 16, 16)
    t3 = _toeplitz(conv3_w, 8, 8)
    b1t = jnp.tile(conv1_b, (1, 16))
    b2t = jnp.tile(conv2_b, (1, 8))
    b3t = jnp.tile(conv3_b, (1, 4))

    fw1 = fc1_w.astype(jnp.bfloat16)
    fw2 = fc2_w.astype(jnp.bfloat16)

    out = pl.pallas_call(
        _net_kernel,
        out_shape=jax.ShapeDtypeStruct((B + Bpad, fw2.shape[1]), jnp.float32),
        grid=(G,),
        in_specs=[
            pl.BlockSpec((34, Bb, 256), lambda i: (0, i, 0)),
            pl.BlockSpec(t1.shape, lambda i: (0, 0)),
            pl.BlockSpec(b1t.shape, lambda i: (0, 0)),
            pl.BlockSpec(t2.shape, lambda i: (0, 0)),
            pl.BlockSpec(b2t.shape, lambda i: (0, 0)),
            pl.BlockSpec(t3.shape, lambda i: (0, 0)),
            pl.BlockSpec(b3t.shape, lambda i: (0, 0)),
            pl.BlockSpec(fw1.shape, lambda i: (0, 0)),
            pl.BlockSpec(fc1_b.shape, lambda i: (0, 0)),
            pl.BlockSpec(fw2.shape, lambda i: (0, 0)),
            pl.BlockSpec(fc2_b.shape, lambda i: (0, 0)),
        ],
        out_specs=pl.BlockSpec((Bb, fw2.shape[1]), lambda i: (i, 0)),
        compiler_params=pltpu.CompilerParams(
            dimension_semantics=("parallel",),
            vmem_limit_bytes=_VMEM_LIMIT),
    )(xm, t1, b1t, t2, b2t, t3, b3t, fw1, fc1_b, fw2, fc2_b)
    return out[:B, :10]


# speedup vs baseline: 1.5856x; 1.0661x over previous
import numpy as np
import jax
import jax.numpy as jnp
from jax.experimental import pallas as pl
from jax.experimental.pallas import tpu as pltpu

_VMEM_LIMIT = 60 * 1024 * 1024
_BB = 128


def _sel(kw, n_in, n_out):
    s = np.zeros((n_in, n_out), np.float32)
    for w in range(n_out):
        p = w + kw - 1
        if 0 <= p < n_in:
            s[p, w] = 1.0
    return s


def _toeplitz(w_taps, n_in, n_out, cw_rows=False):
    cin, cout = w_taps.shape[1], w_taps.shape[2]
    wt = w_taps.reshape(3, 3, cin, cout).astype(jnp.bfloat16)
    perm = np.concatenate([np.arange(0, n_out, 2), np.arange(1, n_out, 2)])
    s = np.stack([_sel(kw, n_in, n_out)[:, perm] for kw in range(3)])
    s = jnp.asarray(s, jnp.bfloat16)
    if cw_rows:
        t = jnp.einsum('kuw,hkcd->hcuwd', s, wt)
    else:
        t = jnp.einsum('kuw,hkcd->hucwd', s, wt)
    return t.reshape(3 * n_in * cin, n_out * cout)


def _net_kernel(x_ref, t1_ref, b1_ref, t2_ref, b2_ref, t3_ref, b3_ref,
                fw1_ref, fb1_ref, fw2_ref, fb2_ref, o_ref):
    Bb = x_ref.shape[1]

    def conv(xp, t_ref, H):
        lhs = jnp.concatenate([xp[0:H], xp[1:H + 1], xp[2:H + 2]], axis=-1)
        acc = jnp.dot(lhs.reshape(H * Bb, lhs.shape[2]), t_ref[...],
                      preferred_element_type=jnp.float32)
        return acc.reshape(H, Bb, t_ref.shape[1])

    def pool_bias_relu_compact(y, bt):
        y4 = y.reshape(y.shape[0] // 2, 2, Bb, y.shape[2])
        yh = jnp.maximum(y4[:, 0], y4[:, 1])
        half = yh.shape[2] // 2
        yw = jnp.maximum(yh[..., :half], yh[..., half:])
        return jnp.maximum(yw + bt, 0.0).astype(jnp.bfloat16)

    b1 = b1_ref[...].reshape(1, 1, -1)
    b2 = b2_ref[...].reshape(1, 1, -1)
    b3 = b3_ref[...].reshape(1, 1, -1)

    h = conv(x_ref[...], t1_ref, 32)
    h = pool_bias_relu_compact(h, b1)
    h = conv(jnp.pad(h, ((1, 1), (0, 0), (0, 0))), t2_ref, 16)
    h = pool_bias_relu_compact(h, b2)
    h = conv(jnp.pad(h, ((1, 1), (0, 0), (0, 0))), t3_ref, 8)
    h = pool_bias_relu_compact(h, b3)

    flat = jnp.concatenate([h[0], h[1], h[2], h[3]], axis=-1)
    z = jnp.dot(flat, fw1_ref[...], preferred_element_type=jnp.float32)
    z = jnp.maximum(z + fb1_ref[...], 0.0).astype(jnp.bfloat16)
    o_ref[...] = jnp.dot(z, fw2_ref[...],
                         preferred_element_type=jnp.float32) + fb2_ref[...]


def kernel(x_nchw, conv1_w, conv1_b, conv2_w, conv2_b, conv3_w, conv3_b,
           fc1_w, fc1_b, fc2_w, fc2_b):
    B = x_nchw.shape[0]
    xm = jnp.transpose(x_nchw, (2, 0, 1, 3)).astype(jnp.bfloat16)
    xm = jnp.pad(xm, ((1, 1), (0, 0), (0, 8 - xm.shape[2]), (0, 0)))
    xm = xm.reshape(34, B, 256)

    Bb = _BB
    Bpad = -B % Bb
    if Bpad:
        xm = jnp.pad(xm, ((0, 0), (0, Bpad), (0, 0)))
    G = (B + Bpad) // Bb

    t1 = _toeplitz(conv1_w, 32, 32, cw_rows=True)
    t2 = _toeplitz(conv2_w, 16, 16)
    t3 = _toeplitz(conv3_w, 8, 8)
    b1t = jnp.tile(conv1_b, (1, 16))
    b2t = jnp.tile(conv2_b, (1, 8))
    b3t = jnp.tile(conv3_b, (1, 4))

    fw1 = fc1_w.astype(jnp.bfloat16)
    fw2 = fc2_w.astype(jnp.bfloat16)

    out = pl.pallas_call(
        _net_kernel,
        out_shape=jax.ShapeDtypeStruct((B + Bpad, fw2.shape[1]), jnp.float32),
        grid=(G,),
        in_specs=[
            pl.BlockSpec((34, Bb, 256), lambda i: (0, i, 0)),
            pl.BlockSpec(t1.shape, lambda i: (0, 0)),
            pl.BlockSpec(b1t.shape, lambda i: (0, 0)),
            pl.BlockSpec(t2.shape, lambda i: (0, 0)),
            pl.BlockSpec(b2t.shape, lambda i: (0, 0)),
            pl.BlockSpec(t3.shape, lambda i: (0, 0)),
            pl.BlockSpec(b3t.shape, lambda i: (0, 0)),
            pl.BlockSpec(fw1.shape, lambda i: (0, 0)),
            pl.BlockSpec(fc1_b.shape, lambda i: (0, 0)),
            pl.BlockSpec(fw2.shape, lambda i: (0, 0)),
            pl.BlockSpec(fc2_b.shape, lambda i: (0, 0)),
        ],
        out_specs=pl.BlockSpec((Bb, fw2.shape[1]), lambda i: (i, 0)),
        compiler_params=pltpu.CompilerParams(
            dimension_semantics=("parallel",),
            vmem_limit_bytes=_VMEM_LIMIT),
    )(xm, t1, b1t, t2, b2t, t3, b3t, fw1, fc1_b, fw2, fc2_b)
    return out[:B, :10]
